# baseline (speedup 1.0000x reference)
import functools

import jax
import jax.numpy as jnp
from jax import lax
from jax.experimental import pallas as pl
from jax.experimental.pallas import tpu as pltpu
from jax.experimental.pallas import tpu_sc as plsc

N_NODES = 10000
N_EDGES = 320000
D_X = 128
D_E = 16
D_U = 16
N_GRAPHS = 16
H = 128

NC = 2
NS = 16
NW = NC * NS
EPW = N_EDGES // NW
CH = 80
NCH = EPW // CH
SUP = 2000
NSUP = EPW // SUP
CH_PER_SUP = SUP // CH
NPAD = 10240
NPT = NPAD // NS

BN = 1000
GRID = N_NODES // BN


def _sc_scatter_body(attr_h, ei_h, ones_h, zeros_h, zeros1_h, esum_h, cnt_h,
                     idx_v, upd_v0, upd_v1, ones_v, acc_e, acc_c,
                     sem_l0, sem_l1, sem_e, sem_c):
    c = lax.axis_index("c")
    s = lax.axis_index("s")
    wid = s * NC + c
    base = wid * EPW

    pltpu.sync_copy(zeros_h.at[pl.ds(s * NPT, NPT)], acc_e.at[pl.ds(s * NPT, NPT)])
    pltpu.sync_copy(zeros1_h.at[pl.ds(s * NPT, NPT)], acc_c.at[pl.ds(s * NPT, NPT)])
    pltpu.sync_copy(ones_h, ones_v)
    pltpu.sync_copy(ei_h.at[1, pl.ds(base, EPW)], idx_v)
    plsc.subcore_barrier()

    bufs = (upd_v0, upd_v1)
    sems = (sem_l0, sem_l1)
    loads = [None, None]
    loads[0] = pltpu.async_copy(attr_h.at[pl.ds(base, SUP)], upd_v0, sem_l0)
    for sup in range(NSUP):
        cur = bufs[sup % 2]
        loads[sup % 2].wait()
        if sup + 1 < NSUP:
            loads[(sup + 1) % 2] = pltpu.async_copy(
                attr_h.at[pl.ds(base + (sup + 1) * SUP, SUP)],
                bufs[(sup + 1) % 2], sems[(sup + 1) % 2])

        def inner(k, carry, sup=sup, cur=cur):
            j = sup * CH_PER_SUP + k
            ce = pltpu.async_copy(cur.at[pl.ds(k * CH, CH)],
                                  acc_e.at[idx_v.at[pl.ds(j * CH, CH)]], sem_e, add=True)
            cc = pltpu.async_copy(ones_v, acc_c.at[idx_v.at[pl.ds(j * CH, CH)]], sem_c, add=True)
            ce.wait()
            cc.wait()
            return carry

        lax.fori_loop(0, CH_PER_SUP, inner, 0)

    plsc.subcore_barrier()
    pltpu.sync_copy(acc_e.at[pl.ds(s * NPT, NPT)], esum_h.at[c, pl.ds(s * NPT, NPT)])
    pltpu.sync_copy(acc_c.at[pl.ds(s * NPT, NPT)], cnt_h.at[c, pl.ds(s * NPT, NPT)])


_sc_scatter = functools.partial(
    pl.kernel,
    mesh=plsc.VectorSubcoreMesh(core_axis_name="c", subcore_axis_name="s"),
    out_type=[
        jax.ShapeDtypeStruct((NC, NPAD, D_E), jnp.float32),
        jax.ShapeDtypeStruct((NC, NPAD), jnp.float32),
    ],
    scratch_types=[
        pltpu.VMEM((EPW,), jnp.int32),
        pltpu.VMEM((SUP, D_E), jnp.float32),
        pltpu.VMEM((SUP, D_E), jnp.float32),
        pltpu.VMEM((CH,), jnp.float32),
        pltpu.VMEM_SHARED((NPAD, D_E), jnp.float32),
        pltpu.VMEM_SHARED((NPAD,), jnp.float32),
        pltpu.SemaphoreType.DMA,
        pltpu.SemaphoreType.DMA,
        pltpu.SemaphoreType.DMA,
        pltpu.SemaphoreType.DMA,
    ],
    compiler_params=pltpu.CompilerParams(use_tc_tiling_on_sc=False),
)(_sc_scatter_body)


def _tc_mlp_body(x_ref, es_ref, cn_ref, b_ref, u_ref, w1x_ref, w1e_ref,
                 w1u_ref, b1_ref, w2_ref, b2_ref, o_ref):
    es = es_ref[0] + es_ref[1]
    cn = cn_ref[0] + cn_ref[1]
    e_agg = es / jnp.maximum(cn, 1.0)

    ub = jnp.dot(u_ref[...], w1u_ref[...], preferred_element_type=jnp.float32)
    gi = lax.broadcasted_iota(jnp.int32, (BN, N_GRAPHS), 1)
    oh = (b_ref[...] == gi).astype(jnp.float32)

    h = (jnp.dot(x_ref[...], w1x_ref[...], preferred_element_type=jnp.float32)
         + jnp.dot(e_agg, w1e_ref[...], preferred_element_type=jnp.float32)
         + jnp.dot(oh, ub, preferred_element_type=jnp.float32)
         + b1_ref[...])
    h = jnp.maximum(h, 0.0)
    o_ref[...] = jnp.dot(h, w2_ref[...], preferred_element_type=jnp.float32) + b2_ref[...]


def _tc_mlp(x, esum, cnt, batch2, u, W1x, W1e, W1u, b1r, W2, b2r):
    return pl.pallas_call(
        _tc_mlp_body,
        grid=(GRID,),
        in_specs=[
            pl.BlockSpec((BN, D_X), lambda i: (i, 0)),
            pl.BlockSpec((NC, BN, D_E), lambda i: (0, i, 0)),
            pl.BlockSpec((NC, BN, 1), lambda i: (0, i, 0)),
            pl.BlockSpec((BN, 1), lambda i: (i, 0)),
            pl.BlockSpec((N_GRAPHS, D_U), lambda i: (0, 0)),
            pl.BlockSpec((D_X, H), lambda i: (0, 0)),
            pl.BlockSpec((D_E, H), lambda i: (0, 0)),
            pl.BlockSpec((D_U, H), lambda i: (0, 0)),
            pl.BlockSpec((1, H), lambda i: (0, 0)),
            pl.BlockSpec((H, D_X), lambda i: (0, 0)),
            pl.BlockSpec((1, D_X), lambda i: (0, 0)),
        ],
        out_specs=pl.BlockSpec((BN, D_X), lambda i: (i, 0)),
        out_shape=jax.ShapeDtypeStruct((N_NODES, D_X), jnp.float32),
    )(x, esum, cnt, batch2, u, W1x, W1e, W1u, b1r, W2, b2r)


def kernel(x, edge_index, edge_attr, u, batch, W1, b1, W2, b2):
    ones = jnp.ones((CH,), jnp.float32)
    zeros = jnp.zeros((NPAD, D_E), jnp.float32)
    zeros1 = jnp.zeros((NPAD,), jnp.float32)

    esum, cnt = _sc_scatter(edge_attr, edge_index, ones, zeros, zeros1)
    cnt = cnt.reshape(NC, NPAD, 1)

    batch2 = batch.reshape(N_NODES, 1)
    W1x = W1[:D_X]
    W1e = W1[D_X:D_X + D_E]
    W1u = W1[D_X + D_E:]
    return _tc_mlp(x, esum, cnt, batch2, u, W1x, W1e, W1u,
                   b1.reshape(1, H), W2, b2.reshape(1, D_X))


# speedup vs baseline: 1.0605x; 1.0605x over previous
import functools

import jax
import jax.numpy as jnp
from jax import lax
from jax.experimental import pallas as pl
from jax.experimental.pallas import tpu as pltpu
from jax.experimental.pallas import tpu_sc as plsc

N_NODES = 10000
N_EDGES = 320000
D_X = 128
D_E = 16
D_U = 16
N_GRAPHS = 16
H = 128

NC = 2
NS = 16
NW = NC * NS
EPW = N_EDGES // NW
CH = 80
NCH = EPW // CH
SUP = 2000
NSUP = EPW // SUP
CH_PER_SUP = SUP // CH
NPAD = 10240
NPT = NPAD // NS

BN = 1000
GRID = N_NODES // BN


def _sc_scatter_body(attr_h, dst_h, ones_h, zeros_h, zeros1_h, esum_h, cnt_h,
                     idx_v, upd_v0, upd_v1, ones_v, acc_e, acc_c,
                     sem_l0, sem_l1, sem_e, sem_c):
    c = lax.axis_index("c")
    s = lax.axis_index("s")
    wid = s * NC + c
    base = wid * EPW

    pltpu.sync_copy(zeros_h, acc_e.at[pl.ds(s * NPT, NPT)])
    pltpu.sync_copy(zeros1_h, acc_c.at[pl.ds(s * NPT, NPT)])
    pltpu.sync_copy(ones_h, ones_v)
    pltpu.sync_copy(dst_h.at[pl.ds(base, EPW)], idx_v)
    plsc.subcore_barrier()

    bufs = (upd_v0, upd_v1)
    sems = (sem_l0, sem_l1)
    loads = [None, None]
    loads[0] = pltpu.async_copy(attr_h.at[pl.ds(base, SUP)], upd_v0, sem_l0)
    for sup in range(NSUP):
        cur = bufs[sup % 2]
        loads[sup % 2].wait()
        if sup + 1 < NSUP:
            loads[(sup + 1) % 2] = pltpu.async_copy(
                attr_h.at[pl.ds(base + (sup + 1) * SUP, SUP)],
                bufs[(sup + 1) % 2], sems[(sup + 1) % 2])

        def inner(k, carry, sup=sup, cur=cur):
            j = sup * CH_PER_SUP + k
            ce = pltpu.async_copy(cur.at[pl.ds(k * CH, CH)],
                                  acc_e.at[idx_v.at[pl.ds(j * CH, CH)]], sem_e, add=True)
            cc = pltpu.async_copy(ones_v, acc_c.at[idx_v.at[pl.ds(j * CH, CH)]], sem_c, add=True)
            ce.wait()
            cc.wait()
            return carry

        lax.fori_loop(0, CH_PER_SUP, inner, 0)

    plsc.subcore_barrier()
    pltpu.sync_copy(acc_e.at[pl.ds(s * NPT, NPT)], esum_h.at[c, pl.ds(s * NPT, NPT)])
    pltpu.sync_copy(acc_c.at[pl.ds(s * NPT, NPT)], cnt_h.at[c, s])


_sc_scatter = functools.partial(
    pl.kernel,
    mesh=plsc.VectorSubcoreMesh(core_axis_name="c", subcore_axis_name="s"),
    out_type=[
        jax.ShapeDtypeStruct((NC, NPAD, D_E), jnp.float32),
        jax.ShapeDtypeStruct((NC, NS, NPT), jnp.float32),
    ],
    scratch_types=[
        pltpu.VMEM((EPW,), jnp.int32),
        pltpu.VMEM((SUP, D_E), jnp.float32),
        pltpu.VMEM((SUP, D_E), jnp.float32),
        pltpu.VMEM((CH,), jnp.float32),
        pltpu.VMEM_SHARED((NPAD, D_E), jnp.float32),
        pltpu.VMEM_SHARED((NPAD,), jnp.float32),
        pltpu.SemaphoreType.DMA,
        pltpu.SemaphoreType.DMA,
        pltpu.SemaphoreType.DMA,
        pltpu.SemaphoreType.DMA,
    ],
    compiler_params=pltpu.CompilerParams(use_tc_tiling_on_sc=False),
)(_sc_scatter_body)


def _tc_mlp_body(x_ref, es_ref, cn_ref, oh_ref, u_ref, w1x_ref, w1e_ref,
                 w1u_ref, b1_ref, w2_ref, b2_ref, o_ref):
    i = pl.program_id(0)
    es = es_ref[0] + es_ref[1]
    cn = cn_ref[0, i] + cn_ref[1, i]
    e_agg = es / jnp.maximum(cn.reshape(BN, 1), 1.0)

    ub = jnp.dot(u_ref[...], w1u_ref[...], preferred_element_type=jnp.float32)

    h = (jnp.dot(x_ref[...], w1x_ref[...], preferred_element_type=jnp.float32)
         + jnp.dot(e_agg, w1e_ref[...], preferred_element_type=jnp.float32)
         + jnp.dot(oh_ref[...], ub, preferred_element_type=jnp.float32)
         + b1_ref[...])
    h = jnp.maximum(h, 0.0)
    o_ref[...] = jnp.dot(h, w2_ref[...], preferred_element_type=jnp.float32) + b2_ref[...]


def _tc_mlp(x, esum, cnt, oh, u, W1x, W1e, W1u, b1r, W2, b2r):
    return pl.pallas_call(
        _tc_mlp_body,
        grid=(GRID,),
        in_specs=[
            pl.BlockSpec((BN, D_X), lambda i: (i, 0)),
            pl.BlockSpec((NC, BN, D_E), lambda i: (0, i, 0)),
            pl.BlockSpec((NC, GRID, BN), lambda i: (0, 0, 0)),
            pl.BlockSpec((BN, N_GRAPHS), lambda i: (i, 0)),
            pl.BlockSpec((N_GRAPHS, D_U), lambda i: (0, 0)),
            pl.BlockSpec((D_X, H), lambda i: (0, 0)),
            pl.BlockSpec((D_E, H), lambda i: (0, 0)),
            pl.BlockSpec((D_U, H), lambda i: (0, 0)),
            pl.BlockSpec((1, H), lambda i: (0, 0)),
            pl.BlockSpec((H, D_X), lambda i: (0, 0)),
            pl.BlockSpec((1, D_X), lambda i: (0, 0)),
        ],
        out_specs=pl.BlockSpec((BN, D_X), lambda i: (i, 0)),
        out_shape=jax.ShapeDtypeStruct((N_NODES, D_X), jnp.float32),
    )(x, esum, cnt, oh, u, W1x, W1e, W1u, b1r, W2, b2r)


def kernel(x, edge_index, edge_attr, u, batch, W1, b1, W2, b2):
    dst = edge_index[1]
    ones = jnp.ones((CH,), jnp.float32)
    zeros = jnp.zeros((NPT, D_E), jnp.float32)
    zeros1 = jnp.zeros((NPT,), jnp.float32)

    esum, cnt = _sc_scatter(edge_attr, dst, ones, zeros, zeros1)
    cnt = cnt.reshape(NC, NPAD)[:, :N_NODES].reshape(NC, GRID, BN)

    oh = jax.nn.one_hot(batch, N_GRAPHS, dtype=jnp.float32)
    W1x = W1[:D_X]
    W1e = W1[D_X:D_X + D_E]
    W1u = W1[D_X + D_E:]
    return _tc_mlp(x, esum, cnt, oh, u, W1x, W1e, W1u,
                   b1.reshape(1, H), W2, b2.reshape(1, D_X))
